# Initial kernel scaffold; baseline (speedup 1.0000x reference)
import jax
import jax.numpy as jnp
from jax.experimental import pallas as pl


def kernel(y, exp_map, kernel, center_kernel, bias):
    raise NotImplementedError("write your pallas kernel here")


# speedup vs baseline: 6.1631x; 6.1631x over previous
import functools

import jax
import jax.numpy as jnp
from jax import lax
from jax.experimental import pallas as pl
from jax.experimental.pallas import tpu as pltpu
from jax.experimental.pallas import tpu_sc as plsc

_NB, _NV, _NR, _ND, _NC, _NF = 1, 10000, 4, 8, 128, 64
_NIDX = _NB * _NV * _NR * _ND
_CH = 128
_NCHUNK = _NIDX // _CH
_NW = 32
_CPW = -(-_NCHUNK // _NW)

_VB = 400
_KD = _NR * _ND * _NC


def _sc_gather(table, idx):
    @functools.partial(
        pl.kernel,
        mesh=plsc.VectorSubcoreMesh(core_axis_name="c", subcore_axis_name="s"),
        out_type=jax.ShapeDtypeStruct((_NIDX, _NC), jnp.float32),
        scratch_types=[
            pltpu.VMEM((_CH,), jnp.int32),
            pltpu.VMEM((_CH, _NC), jnp.float32),
            pltpu.SemaphoreType.DMA,
        ],
    )
    def gather_kernel(table_hbm, idx_hbm, out_hbm, idx_v, rows_v, sem):
        cid = lax.axis_index("c")
        sid = lax.axis_index("s")
        wid = sid * 2 + cid

        def body(i, _):
            chunk = wid + i * _NW

            @pl.when(chunk < _NCHUNK)
            def _():
                base = chunk * _CH
                pltpu.sync_copy(idx_hbm.at[pl.ds(base, _CH)], idx_v)
                pltpu.async_copy(table_hbm.at[idx_v], rows_v, sem).wait()
                pltpu.sync_copy(rows_v, out_hbm.at[pl.ds(base, _CH)])

            return ()

        lax.fori_loop(0, _CPW, body, ())

    return gather_kernel(table, idx)


def _tc_body(g_ref, y_ref, w_ref, ck_ref, b_ref, o_ref):
    acc = jnp.dot(g_ref[...], w_ref[...], preferred_element_type=jnp.float32)
    m = jnp.maximum(acc[:, : 4 * _NF], acc[:, 4 * _NF :])
    m = jnp.maximum(m[:, : 2 * _NF], m[:, 2 * _NF :])
    m = jnp.maximum(m[:, :_NF], m[:, _NF:])
    cent = jnp.dot(y_ref[...], ck_ref[...], preferred_element_type=jnp.float32)
    o_ref[...] = jnp.maximum(m + cent + b_ref[...], 0.0)


def _tc_conv(g2, y2, wbig, ck, bias2):
    nvb = _NB * _NV
    grid = (nvb // _VB,)
    return pl.pallas_call(
        _tc_body,
        grid=grid,
        in_specs=[
            pl.BlockSpec((_VB, _KD), lambda i: (i, 0)),
            pl.BlockSpec((_VB, _NC), lambda i: (i, 0)),
            pl.BlockSpec((_KD, _ND * _NF), lambda i: (0, 0)),
            pl.BlockSpec((_NC, _NF), lambda i: (0, 0)),
            pl.BlockSpec((1, _NF), lambda i: (0, 0)),
        ],
        out_specs=pl.BlockSpec((_VB, _NF), lambda i: (i, 0)),
        out_shape=jax.ShapeDtypeStruct((nvb, _NF), jnp.float32),
    )(g2, y2, wbig, ck, bias2)


def kernel(y, exp_map, kernel, center_kernel, bias):
    nb, nv, nc = y.shape
    nr, nd, _, nf = kernel.shape
    y2 = y.reshape(nb * nv, nc)
    idx = (exp_map[..., 0] * nv + exp_map[..., 1]).reshape(-1).astype(jnp.int32)
    g = _sc_gather(y2, idx)
    wbig = jnp.concatenate(
        [jnp.roll(kernel, d, axis=1).reshape(nr * nd * nc, nf) for d in range(nd)],
        axis=1,
    )
    out = _tc_conv(g.reshape(nb * nv, nr * nd * nc), y2, wbig,
                   center_kernel, bias.reshape(1, nf))
    return out.reshape(nb, nv, nf)


# speedup vs baseline: 7.4505x; 1.2089x over previous
import functools

import jax
import jax.numpy as jnp
from jax import lax
from jax.experimental import pallas as pl
from jax.experimental.pallas import tpu as pltpu
from jax.experimental.pallas import tpu_sc as plsc

_NB, _NV, _NR, _ND, _NC, _NF = 1, 10000, 4, 8, 128, 64
_NIDX = _NB * _NV * _NR * _ND
_NW = 32
_RPW = _NIDX // _NW
_CH = 80
_NCH = _RPW // _CH
_RING = 5
_NGRP = _NCH // _RING

_VB = 400
_KD = _NR * _ND * _NC


def _sc_gather(table, idx):
    width = table.shape[1]

    @functools.partial(
        pl.kernel,
        mesh=plsc.VectorSubcoreMesh(core_axis_name="c", subcore_axis_name="s"),
        out_type=jax.ShapeDtypeStruct((_NIDX, width), table.dtype),
        scratch_types=[pltpu.VMEM((_RPW,), jnp.int32)]
        + [pltpu.VMEM((_CH, width), table.dtype) for _ in range(_RING)]
        + [pltpu.SemaphoreType.DMA for _ in range(_RING)]
        + [pltpu.SemaphoreType.DMA],
    )
    def gather_kernel(table_hbm, idx_hbm, out_hbm, idx_v, *bufs_sems):
        rows = bufs_sems[:_RING]
        gsem = bufs_sems[_RING:2 * _RING]
        wsem = bufs_sems[2 * _RING]
        cid = lax.axis_index("c")
        sid = lax.axis_index("s")
        wid = sid * 2 + cid
        base = wid * _RPW
        pltpu.sync_copy(idx_hbm.at[pl.ds(base, _RPW)], idx_v)

        def gather_copy(chunk, b):
            return pltpu.make_async_copy(
                table_hbm.at[idx_v.at[pl.ds(chunk * _CH, _CH)]], rows[b], gsem[b])

        for b in range(_RING):
            gather_copy(b, b).start()

        def body(g, _):
            for b in range(_RING):
                chunk = g * _RING + b
                gather_copy(chunk, b).wait()
                wcopy = pltpu.make_async_copy(
                    rows[b], out_hbm.at[pl.ds(base + chunk * _CH, _CH)], wsem)
                wcopy.start()
                wcopy.wait()
                nxt = chunk + _RING

                @pl.when(nxt < _NCH)
                def _():
                    gather_copy(nxt, b).start()

            return ()

        lax.fori_loop(0, _NGRP, body, ())

    return gather_kernel(table, idx)


def _tc_body(g_ref, y_ref, w_ref, ck_ref, b_ref, o_ref):
    acc = jnp.dot(g_ref[...], w_ref[...], preferred_element_type=jnp.float32)
    m = jnp.maximum(acc[:, : 4 * _NF], acc[:, 4 * _NF :])
    m = jnp.maximum(m[:, : 2 * _NF], m[:, 2 * _NF :])
    m = jnp.maximum(m[:, :_NF], m[:, _NF:])
    cent = jnp.dot(y_ref[...], ck_ref[...], preferred_element_type=jnp.float32)
    o_ref[...] = jnp.maximum(m + cent + b_ref[...], 0.0)


def _tc_conv(g2, y2, wbig, ck, bias2):
    nvb = _NB * _NV
    grid = (nvb // _VB,)
    return pl.pallas_call(
        _tc_body,
        grid=grid,
        in_specs=[
            pl.BlockSpec((_VB, _KD), lambda i: (i, 0)),
            pl.BlockSpec((_VB, _NC), lambda i: (i, 0)),
            pl.BlockSpec((_KD, _ND * _NF), lambda i: (0, 0)),
            pl.BlockSpec((_NC, _NF), lambda i: (0, 0)),
            pl.BlockSpec((1, _NF), lambda i: (0, 0)),
        ],
        out_specs=pl.BlockSpec((_VB, _NF), lambda i: (i, 0)),
        out_shape=jax.ShapeDtypeStruct((nvb, _NF), jnp.float32),
    )(g2, y2, wbig, ck, bias2)


def kernel(y, exp_map, kernel, center_kernel, bias):
    nb, nv, nc = y.shape
    nr, nd, _, nf = kernel.shape
    y2 = y.reshape(nb * nv, nc)
    idx = (exp_map[..., 0] * nv + exp_map[..., 1]).reshape(-1).astype(jnp.int32)
    g = _sc_gather(y2, idx)
    wbig = jnp.concatenate(
        [jnp.roll(kernel, d, axis=1).reshape(nr * nd * nc, nf) for d in range(nd)],
        axis=1,
    )
    out = _tc_conv(g.reshape(nb * nv, nr * nd * nc), y2, wbig,
                   center_kernel, bias.reshape(1, nf))
    return out.reshape(nb, nv, nf)


# speedup vs baseline: 7.6773x; 1.0304x over previous
import functools

import jax
import jax.numpy as jnp
from jax import lax
from jax.experimental import pallas as pl
from jax.experimental.pallas import tpu as pltpu
from jax.experimental.pallas import tpu_sc as plsc

_NB, _NV, _NR, _ND, _NC, _NF = 1, 10000, 4, 8, 128, 64
_NIDX = _NB * _NV * _NR * _ND
_NW = 32
_RPW = _NIDX // _NW
_CH = 80
_NCH = _RPW // _CH
_RING = 5
_NGRP = _NCH // _RING

_VB = 400
_KD = _NR * _ND * _NC


def _sc_gather(table, idx):
    width = table.shape[1]

    @functools.partial(
        pl.kernel,
        mesh=plsc.VectorSubcoreMesh(core_axis_name="c", subcore_axis_name="s"),
        out_type=jax.ShapeDtypeStruct((_NIDX, width), table.dtype),
        scratch_types=[pltpu.VMEM((_RPW,), jnp.int32)]
        + [pltpu.VMEM((_CH, width), table.dtype) for _ in range(_RING)]
        + [pltpu.SemaphoreType.DMA for _ in range(_RING)]
        + [pltpu.SemaphoreType.DMA],
    )
    def gather_kernel(table_hbm, idx_hbm, out_hbm, idx_v, *bufs_sems):
        rows = bufs_sems[:_RING]
        gsem = bufs_sems[_RING:2 * _RING]
        wsem = bufs_sems[2 * _RING]
        cid = lax.axis_index("c")
        sid = lax.axis_index("s")
        wid = sid * 2 + cid
        base = wid * _RPW
        pltpu.sync_copy(idx_hbm.at[pl.ds(base, _RPW)], idx_v)

        def gather_copy(chunk, b):
            return pltpu.make_async_copy(
                table_hbm.at[idx_v.at[pl.ds(chunk * _CH, _CH)]], rows[b], gsem[b])

        for b in range(_RING):
            gather_copy(b, b).start()

        def body(g, _):
            for b in range(_RING):
                chunk = g * _RING + b
                gather_copy(chunk, b).wait()
                wcopy = pltpu.make_async_copy(
                    rows[b], out_hbm.at[pl.ds(base + chunk * _CH, _CH)], wsem)
                wcopy.start()
                wcopy.wait()
                nxt = chunk + _RING

                @pl.when(nxt < _NCH)
                def _():
                    gather_copy(nxt, b).start()

            return ()

        lax.fori_loop(0, _NGRP, body, ())

    return gather_kernel(table, idx)


def _tc_body(g_ref, y_ref, w_ref, ck_ref, b_ref, o_ref):
    acc = jnp.dot(g_ref[...].astype(jnp.bfloat16), w_ref[...],
                  preferred_element_type=jnp.float32)
    m = jnp.maximum(acc[:, : 4 * _NF], acc[:, 4 * _NF :])
    m = jnp.maximum(m[:, : 2 * _NF], m[:, 2 * _NF :])
    m = jnp.maximum(m[:, :_NF], m[:, _NF:])
    cent = jnp.dot(y_ref[...], ck_ref[...], preferred_element_type=jnp.float32)
    o_ref[...] = jnp.maximum(m + cent + b_ref[...], 0.0)


def _tc_conv(g2, y2, wbig, ck, bias2):
    nvb = _NB * _NV
    grid = (nvb // _VB,)
    return pl.pallas_call(
        _tc_body,
        grid=grid,
        in_specs=[
            pl.BlockSpec((_VB, _KD), lambda i: (i, 0)),
            pl.BlockSpec((_VB, _NC), lambda i: (i, 0)),
            pl.BlockSpec((_KD, _ND * _NF), lambda i: (0, 0)),
            pl.BlockSpec((_NC, _NF), lambda i: (0, 0)),
            pl.BlockSpec((1, _NF), lambda i: (0, 0)),
        ],
        out_specs=pl.BlockSpec((_VB, _NF), lambda i: (i, 0)),
        out_shape=jax.ShapeDtypeStruct((nvb, _NF), jnp.float32),
    )(g2, y2, wbig, ck, bias2)


def kernel(y, exp_map, kernel, center_kernel, bias):
    nb, nv, nc = y.shape
    nr, nd, _, nf = kernel.shape
    y2 = y.reshape(nb * nv, nc)
    idx = (exp_map[..., 0] * nv + exp_map[..., 1]).reshape(-1).astype(jnp.int32)
    g = _sc_gather(y2, idx).reshape(nb * nv, nr * nd * nc)
    wbig = jnp.concatenate(
        [jnp.roll(kernel, d, axis=1).reshape(nr * nd * nc, nf) for d in range(nd)],
        axis=1,
    ).astype(jnp.bfloat16)
    out = _tc_conv(g, y2, wbig, center_kernel, bias.reshape(1, nf))
    return out.reshape(nb, nv, nf)


# speedup vs baseline: 12.9154x; 1.6823x over previous
import functools

import jax
import jax.numpy as jnp
from jax import lax
from jax.experimental import pallas as pl
from jax.experimental.pallas import tpu as pltpu
from jax.experimental.pallas import tpu_sc as plsc

_NB, _NV, _NR, _ND, _NC, _NF = 1, 10000, 4, 8, 128, 64
_NIDX = _NB * _NV * _NR * _ND
_NW = 32
_RPW = _NIDX // _NW
_CH = 80
_NCH = _RPW // _CH
_RING = 5
_NGRP = _NCH // _RING

_VB = 400
_RJ = _NR * _ND


def _sc_gather(table, idx):
    width = table.shape[1]

    @functools.partial(
        pl.kernel,
        mesh=plsc.VectorSubcoreMesh(core_axis_name="c", subcore_axis_name="s"),
        out_type=jax.ShapeDtypeStruct((_NIDX, width), table.dtype),
        scratch_types=[pltpu.VMEM((_RPW,), jnp.int32)]
        + [pltpu.VMEM((_CH, width), table.dtype) for _ in range(_RING)]
        + [pltpu.SemaphoreType.DMA for _ in range(_RING)]
        + [pltpu.SemaphoreType.DMA],
    )
    def gather_kernel(table_hbm, idx_hbm, out_hbm, idx_v, *bufs_sems):
        rows = bufs_sems[:_RING]
        gsem = bufs_sems[_RING:2 * _RING]
        wsem = bufs_sems[2 * _RING]
        cid = lax.axis_index("c")
        sid = lax.axis_index("s")
        wid = sid * 2 + cid
        base = wid * _RPW
        pltpu.sync_copy(idx_hbm.at[pl.ds(base, _RPW)], idx_v)

        def gather_copy(chunk, b):
            return pltpu.make_async_copy(
                table_hbm.at[idx_v.at[pl.ds(chunk * _CH, _CH)]], rows[b], gsem[b])

        for b in range(_RING):
            gather_copy(b, b).start()

        def body(g, _):
            for b in range(_RING):
                chunk = g * _RING + b
                gather_copy(chunk, b).wait()
                wcopy = pltpu.make_async_copy(
                    rows[b], out_hbm.at[pl.ds(base + chunk * _CH, _CH)], wsem)
                wcopy.start()
                wcopy.wait()
                nxt = chunk + _RING

                @pl.when(nxt < _NCH)
                def _():
                    gather_copy(nxt, b).start()

            return ()

        lax.fori_loop(0, _NGRP, body, ())

    return gather_kernel(table, idx)


def _tc_body(g_ref, y_ref, k_ref, ck_ref, b_ref, o_ref, w_s):
    i = pl.program_id(0)

    @pl.when(i == 0)
    def _build_w():
        for rj in range(_RJ):
            r, j = divmod(rj, _ND)
            for d in range(_ND):
                src = r * _ND + (j - d) % _ND
                w_s[rj, :, d * _NF:(d + 1) * _NF] = k_ref[src].astype(jnp.bfloat16)

    acc = jnp.zeros((_VB, _ND * _NF), jnp.float32)
    for rj in range(_RJ):
        acc = acc + jnp.dot(g_ref[rj].astype(jnp.bfloat16), w_s[rj],
                            preferred_element_type=jnp.float32)
    m = jnp.maximum(acc[:, : 4 * _NF], acc[:, 4 * _NF:])
    m = jnp.maximum(m[:, : 2 * _NF], m[:, 2 * _NF:])
    m = jnp.maximum(m[:, :_NF], m[:, _NF:])
    cent = jnp.dot(y_ref[...], ck_ref[...], preferred_element_type=jnp.float32)
    o_ref[...] = jnp.maximum(m + cent + b_ref[...], 0.0)


def _tc_conv(g3, y2, k2, ck, bias2):
    nvb = _NB * _NV
    grid = (nvb // _VB,)
    return pl.pallas_call(
        _tc_body,
        grid=grid,
        in_specs=[
            pl.BlockSpec((_RJ, _VB, _NC), lambda i: (0, i, 0)),
            pl.BlockSpec((_VB, _NC), lambda i: (i, 0)),
            pl.BlockSpec((_RJ, _NC, _NF), lambda i: (0, 0, 0)),
            pl.BlockSpec((_NC, _NF), lambda i: (0, 0)),
            pl.BlockSpec((1, _NF), lambda i: (0, 0)),
        ],
        out_specs=pl.BlockSpec((_VB, _NF), lambda i: (i, 0)),
        out_shape=jax.ShapeDtypeStruct((nvb, _NF), jnp.float32),
        scratch_shapes=[pltpu.VMEM((_RJ, _NC, _ND * _NF), jnp.bfloat16)],
    )(g3, y2, k2, ck, bias2)


def kernel(y, exp_map, kernel, center_kernel, bias):
    nb, nv, nc = y.shape
    nr, nd, _, nf = kernel.shape
    y2 = y.reshape(nb * nv, nc)
    idx = jnp.transpose(
        exp_map[..., 0] * nv + exp_map[..., 1], (0, 2, 3, 1)
    ).reshape(-1).astype(jnp.int32)
    g3 = _sc_gather(y2, idx).reshape(nr * nd, nb * nv, nc)
    out = _tc_conv(g3, y2, kernel.reshape(nr * nd, nc, nf),
                   center_kernel, bias.reshape(1, nf))
    return out.reshape(nb, nv, nf)


# speedup vs baseline: 13.5224x; 1.0470x over previous
import functools

import jax
import jax.numpy as jnp
from jax import lax
from jax.experimental import pallas as pl
from jax.experimental.pallas import tpu as pltpu
from jax.experimental.pallas import tpu_sc as plsc

_NB, _NV, _NR, _ND, _NC, _NF = 1, 10000, 4, 8, 128, 64
_NW = 32
_CH = 80
_RING = 5

_NPIECE = 5
_P = _NV // _NPIECE
_VB = 400
_RJ = _NR * _ND


def _sc_gather(table, idx):
    width = table.shape[1]
    nidx = idx.shape[0]
    rpw = nidx // _NW
    nch = rpw // _CH
    ngrp = nch // _RING
    assert rpw % _CH == 0 and nch % _RING == 0

    @functools.partial(
        pl.kernel,
        mesh=plsc.VectorSubcoreMesh(core_axis_name="c", subcore_axis_name="s"),
        out_type=jax.ShapeDtypeStruct((nidx, width), table.dtype),
        scratch_types=[pltpu.VMEM((rpw,), jnp.int32)]
        + [pltpu.VMEM((_CH, width), table.dtype) for _ in range(_RING)]
        + [pltpu.SemaphoreType.DMA for _ in range(_RING)]
        + [pltpu.SemaphoreType.DMA],
    )
    def gather_kernel(table_hbm, idx_hbm, out_hbm, idx_v, *bufs_sems):
        rows = bufs_sems[:_RING]
        gsem = bufs_sems[_RING:2 * _RING]
        wsem = bufs_sems[2 * _RING]
        cid = lax.axis_index("c")
        sid = lax.axis_index("s")
        wid = sid * 2 + cid
        base = wid * rpw
        pltpu.sync_copy(idx_hbm.at[pl.ds(base, rpw)], idx_v)

        def gather_copy(chunk, b):
            return pltpu.make_async_copy(
                table_hbm.at[idx_v.at[pl.ds(chunk * _CH, _CH)]], rows[b], gsem[b])

        for b in range(_RING):
            gather_copy(b, b).start()

        def body(g, _):
            for b in range(_RING):
                chunk = g * _RING + b
                gather_copy(chunk, b).wait()
                wcopy = pltpu.make_async_copy(
                    rows[b], out_hbm.at[pl.ds(base + chunk * _CH, _CH)], wsem)
                wcopy.start()
                wcopy.wait()
                nxt = chunk + _RING

                @pl.when(nxt < nch)
                def _():
                    gather_copy(nxt, b).start()

            return ()

        lax.fori_loop(0, ngrp, body, ())

    return gather_kernel(table, idx)


def _tc_body(g_ref, y_ref, k_ref, ck_ref, b_ref, o_ref, w_s):
    i = pl.program_id(0)

    @pl.when(i == 0)
    def _build_w():
        for rj in range(_RJ):
            r, j = divmod(rj, _ND)
            for d in range(_ND):
                src = r * _ND + (j - d) % _ND
                w_s[rj, :, d * _NF:(d + 1) * _NF] = k_ref[src].astype(jnp.bfloat16)

    acc = jnp.zeros((_VB, _ND * _NF), jnp.float32)
    for rj in range(_RJ):
        acc = acc + jnp.dot(g_ref[rj].astype(jnp.bfloat16), w_s[rj],
                            preferred_element_type=jnp.float32)
    m = jnp.maximum(acc[:, : 4 * _NF], acc[:, 4 * _NF:])
    m = jnp.maximum(m[:, : 2 * _NF], m[:, 2 * _NF:])
    m = jnp.maximum(m[:, :_NF], m[:, _NF:])
    cent = jnp.dot(y_ref[...], ck_ref[...], preferred_element_type=jnp.float32)
    o_ref[...] = jnp.maximum(m + cent + b_ref[...], 0.0)


def _tc_conv(g3, y2, k2, ck, bias2):
    nvp = g3.shape[1]
    grid = (nvp // _VB,)
    return pl.pallas_call(
        _tc_body,
        grid=grid,
        in_specs=[
            pl.BlockSpec((_RJ, _VB, _NC), lambda i: (0, i, 0)),
            pl.BlockSpec((_VB, _NC), lambda i: (i, 0)),
            pl.BlockSpec((_RJ, _NC, _NF), lambda i: (0, 0, 0)),
            pl.BlockSpec((_NC, _NF), lambda i: (0, 0)),
            pl.BlockSpec((1, _NF), lambda i: (0, 0)),
        ],
        out_specs=pl.BlockSpec((_VB, _NF), lambda i: (i, 0)),
        out_shape=jax.ShapeDtypeStruct((nvp, _NF), jnp.float32),
        scratch_shapes=[pltpu.VMEM((_RJ, _NC, _ND * _NF), jnp.bfloat16)],
    )(g3, y2, k2, ck, bias2)


def kernel(y, exp_map, kernel, center_kernel, bias):
    nb, nv, nc = y.shape
    nr, nd, _, nf = kernel.shape
    y2 = y.reshape(nb * nv, nc)
    idx3 = jnp.transpose(
        exp_map[..., 0] * nv + exp_map[..., 1], (0, 2, 3, 1)
    ).reshape(nr * nd, nb * nv).astype(jnp.int32)
    k2 = kernel.reshape(nr * nd, nc, nf)
    bias2 = bias.reshape(1, nf)
    outs = []
    for p in range(_NPIECE):
        sl = slice(p * _P, (p + 1) * _P)
        g3 = _sc_gather(y2, idx3[:, sl].reshape(-1)).reshape(nr * nd, _P, nc)
        outs.append(_tc_conv(g3, y2[sl], k2, center_kernel, bias2))
    out = jnp.concatenate(outs, axis=0)
    return out.reshape(nb, nv, nf)


# speedup vs baseline: 13.8065x; 1.0210x over previous
import functools

import jax
import jax.numpy as jnp
from jax import lax
from jax.experimental import pallas as pl
from jax.experimental.pallas import tpu as pltpu
from jax.experimental.pallas import tpu_sc as plsc

_NB, _NV, _NR, _ND, _NC, _NF = 1, 10000, 4, 8, 128, 64
_NW = 32
_CH = 80
_RING = 5

_NPIECE = 5
_P = _NV // _NPIECE
_VB = 400
_RJ = _NR * _ND


def _sc_gather(table, idx):
    width = table.shape[1]
    nidx = idx.shape[0]
    rpw = nidx // _NW
    nch = rpw // _CH
    ngrp = nch // _RING
    assert rpw % _CH == 0 and nch % _RING == 0

    @functools.partial(
        pl.kernel,
        mesh=plsc.VectorSubcoreMesh(core_axis_name="c", subcore_axis_name="s"),
        out_type=jax.ShapeDtypeStruct((nidx, width), table.dtype),
        scratch_types=[pltpu.VMEM((rpw,), jnp.int32)]
        + [pltpu.VMEM((_CH, width), table.dtype) for _ in range(_RING)]
        + [pltpu.SemaphoreType.DMA for _ in range(_RING)]
        + [pltpu.SemaphoreType.DMA],
    )
    def gather_kernel(table_hbm, idx_hbm, out_hbm, idx_v, *bufs_sems):
        rows = bufs_sems[:_RING]
        gsem = bufs_sems[_RING:2 * _RING]
        wsem = bufs_sems[2 * _RING]
        cid = lax.axis_index("c")
        sid = lax.axis_index("s")
        wid = sid * 2 + cid
        base = wid * rpw
        pltpu.sync_copy(idx_hbm.at[pl.ds(base, rpw)], idx_v)

        def gather_copy(chunk, b):
            return pltpu.make_async_copy(
                table_hbm.at[idx_v.at[pl.ds(chunk * _CH, _CH)]], rows[b], gsem[b])

        for b in range(_RING):
            gather_copy(b, b).start()

        def body(g, _):
            for b in range(_RING):
                chunk = g * _RING + b
                gather_copy(chunk, b).wait()
                wcopy = pltpu.make_async_copy(
                    rows[b], out_hbm.at[pl.ds(base + chunk * _CH, _CH)], wsem)
                wcopy.start()
                wcopy.wait()
                nxt = chunk + _RING

                @pl.when(nxt < nch)
                def _():
                    gather_copy(nxt, b).start()

            return ()

        lax.fori_loop(0, ngrp, body, ())

    return gather_kernel(table, idx)


def _w_build_body(k_ref, w_ref):
    for rj in range(_RJ):
        r, j = divmod(rj, _ND)
        for d in range(_ND):
            src = r * _ND + (j - d) % _ND
            w_ref[rj, :, d * _NF:(d + 1) * _NF] = k_ref[src].astype(jnp.bfloat16)


def _build_w(k2):
    return pl.pallas_call(
        _w_build_body,
        out_shape=jax.ShapeDtypeStruct((_RJ, _NC, _ND * _NF), jnp.bfloat16),
    )(k2)


def _tc_body(g_ref, y_ref, w_ref, ck_ref, b_ref, o_ref):
    acc = jnp.zeros((_VB, _ND * _NF), jnp.float32)
    for rj in range(_RJ):
        acc = acc + jnp.dot(g_ref[rj].astype(jnp.bfloat16), w_ref[rj],
                            preferred_element_type=jnp.float32,
                            precision=lax.Precision.DEFAULT)
    m = jnp.maximum(acc[:, : 4 * _NF], acc[:, 4 * _NF:])
    m = jnp.maximum(m[:, : 2 * _NF], m[:, 2 * _NF:])
    m = jnp.maximum(m[:, :_NF], m[:, _NF:])
    cent = jnp.dot(y_ref[...], ck_ref[...], preferred_element_type=jnp.float32)
    o_ref[...] = jnp.maximum(m + cent + b_ref[...], 0.0)


def _tc_conv(g3, y2, w3, ck, bias2):
    nvp = g3.shape[1]
    grid = (nvp // _VB,)
    return pl.pallas_call(
        _tc_body,
        grid=grid,
        in_specs=[
            pl.BlockSpec((_RJ, _VB, _NC), lambda i: (0, i, 0)),
            pl.BlockSpec((_VB, _NC), lambda i: (i, 0)),
            pl.BlockSpec((_RJ, _NC, _ND * _NF), lambda i: (0, 0, 0)),
            pl.BlockSpec((_NC, _NF), lambda i: (0, 0)),
            pl.BlockSpec((1, _NF), lambda i: (0, 0)),
        ],
        out_specs=pl.BlockSpec((_VB, _NF), lambda i: (i, 0)),
        out_shape=jax.ShapeDtypeStruct((nvp, _NF), jnp.float32),
    )(g3, y2, w3, ck, bias2)


def kernel(y, exp_map, kernel, center_kernel, bias):
    nb, nv, nc = y.shape
    nr, nd, _, nf = kernel.shape
    y2 = y.reshape(nb * nv, nc)
    idx3 = jnp.transpose(
        exp_map[..., 0] * nv + exp_map[..., 1], (0, 2, 3, 1)
    ).reshape(nr * nd, nb * nv).astype(jnp.int32)
    w3 = _build_w(kernel.reshape(nr * nd, nc, nf))
    bias2 = bias.reshape(1, nf)
    outs = []
    for p in range(_NPIECE):
        sl = slice(p * _P, (p + 1) * _P)
        g3 = _sc_gather(y2, idx3[:, sl].reshape(-1)).reshape(nr * nd, _P, nc)
        outs.append(_tc_conv(g3, y2[sl], w3, center_kernel, bias2))
    out = jnp.concatenate(outs, axis=0)
    return out.reshape(nb, nv, nf)
